# baseline (speedup 1.0000x reference)
import jax
import jax.numpy as jnp
from jax.experimental import pallas as pl
from jax.experimental.pallas import tpu as pltpu

_N_FIELDS = 26
_EMB = 64
_DM = 512
_STEPS = _N_FIELDS + 1


def _mlp_body(idxs_ref, uid_blk_ref, cat_blk_ref, w1_ref, b1_ref, w2_ref,
              b2_ref, out_ref, acc_ref):
    i = pl.program_id(0)
    r_u = idxs_ref[0] % 8
    k = jnp.maximum(i, 1)
    r_c = idxs_ref[k] % 8
    row_u = uid_blk_ref[pl.ds(r_u, 1), :]
    row_c = cat_blk_ref[0, pl.ds(r_c, 1), :]
    emb = jnp.where(i == 0, row_u, row_c)
    partial = jnp.dot(emb, w1_ref[...], preferred_element_type=jnp.float32)

    @pl.when(i == 0)
    def _init():
        acc_ref[...] = partial

    @pl.when(i > 0)
    def _accum():
        acc_ref[...] += partial

    @pl.when(i == _STEPS - 1)
    def _finish():
        x = acc_ref[...] + b1_ref[...]
        x = jnp.where(x >= 0, x, 0.01 * x)
        out_ref[...] = (jnp.dot(x, w2_ref[...], preferred_element_type=jnp.float32)
                        + b2_ref[...])


def kernel(uid, onehot_feats, uid_table, cat_tables, W1, b1, W2, b2):
    idxs = jnp.concatenate(
        [uid.astype(jnp.int32), onehot_feats.reshape(-1).astype(jnp.int32)])

    grid_spec = pltpu.PrefetchScalarGridSpec(
        num_scalar_prefetch=1,
        grid=(_STEPS,),
        in_specs=[
            pl.BlockSpec((8, _EMB), lambda i, idxs: (idxs[0] // 8, 0)),
            pl.BlockSpec(
                (1, 8, _EMB),
                lambda i, idxs: (jnp.maximum(i, 1) - 1,
                                 idxs[jnp.maximum(i, 1)] // 8, 0)),
            pl.BlockSpec((_EMB, _DM), lambda i, idxs: (i, 0)),
            pl.BlockSpec((1, _DM), lambda i, idxs: (0, 0)),
            pl.BlockSpec((_DM, _DM), lambda i, idxs: (0, 0)),
            pl.BlockSpec((1, _DM), lambda i, idxs: (0, 0)),
        ],
        out_specs=pl.BlockSpec((1, _DM), lambda i, idxs: (0, 0)),
        scratch_shapes=[pltpu.VMEM((1, _DM), jnp.float32)],
    )

    out = pl.pallas_call(
        _mlp_body,
        grid_spec=grid_spec,
        out_shape=jax.ShapeDtypeStruct((1, _DM), jnp.float32),
    )(idxs, uid_table, cat_tables, W1, b1.reshape(1, -1), W2,
      b2.reshape(1, -1))
    return out[None]
